# baseline (speedup 1.0000x reference)
import jax
import jax.numpy as jnp
from jax import lax
from jax.experimental import pallas as pl
from jax.experimental.pallas import tpu as pltpu
from jax.experimental.pallas import tpu_sc as plsc


def _lane_take(v, idx):
    return v.at[idx].get(mode="promise_in_bounds")


def _transpose16(vs, iota):
    for s in (1, 2, 4, 8):
        perm = iota ^ s
        nv = []
        for i in range(16):
            pp = _lane_take(vs[i ^ s], perm)
            keep = (iota & s) == (i & s)
            nv.append(jnp.where(keep, vs[i], pp))
        vs = nv
    return vs


def _pos_embed_body(rw_hbm, cw_hbm, out_hbm, rw_v, cw_v, slab_v, pos_sh, sem):
    cid = lax.axis_index("c")
    sid = lax.axis_index("s")

    pltpu.sync_copy(rw_hbm, rw_v)
    pltpu.sync_copy(cw_hbm, cw_v)

    iota16 = lax.iota(jnp.int32, 16)


    @pl.when(sid < 8)
    def _build_x():
        c0 = sid * 32
        for g in range(2):
            cg = c0 + g * 16
            for xg in range(2):
                vs = [
                    rw_v[pl.ds((xg * 16 + x) * 256 + cg, 16)]
                    for x in range(16)
                ]
                t = _transpose16(vs, iota16)

                def row(y, _, t=t, g=g, xg=xg):
                    for j in range(16):
                        off = (g * 16 + j) * 1024 + y * 32 + xg * 16
                        slab_v[pl.ds(off, 16)] = t[j]
                    return 0

                lax.fori_loop(0, 32, row, 0)

    @pl.when(sid >= 8)
    def _build_y():
        c0 = sid * 32 - 256
        for g in range(2):
            cg = c0 + g * 16
            for yg in range(2):
                vs = [
                    cw_v[pl.ds((yg * 16 + y) * 256 + cg, 16)]
                    for y in range(16)
                ]
                t = _transpose16(vs, iota16)

                def row(y, _, t=t, g=g, yg=yg):
                    lane = jnp.full((16,), y, jnp.int32)
                    for j in range(16):
                        sp = _lane_take(t[j], lane)
                        off = (g * 16 + j) * 1024 + (yg * 16 + y) * 32
                        slab_v[pl.ds(off, 16)] = sp
                        slab_v[pl.ds(off + 16, 16)] = sp
                    return 0

                lax.fori_loop(0, 16, row, 0)

    pltpu.sync_copy(slab_v, pos_sh.at[pl.ds(sid * 32768, 32768)])
    plsc.subcore_barrier()

    @pl.when(sid == 0)
    def _emit():
        handles = [
            pltpu.async_copy(pos_sh, out_hbm.at[cid * 8 + b], sem)
            for b in range(8)
        ]
        for h in handles:
            h.wait()


def kernel(uv_feat, row_weight, col_weight):
    b = uv_feat.shape[0]
    h, w = uv_feat.shape[-2], uv_feat.shape[-1]
    d = row_weight.shape[-1]
    assert (b, h, w, d) == (16, 32, 32, 256)

    mesh = plsc.VectorSubcoreMesh(core_axis_name="c", subcore_axis_name="s")
    run = pl.kernel(
        _pos_embed_body,
        mesh=mesh,
        out_type=jax.ShapeDtypeStruct((b, 2 * d * h * w), jnp.float32),
        scratch_types=[
            pltpu.VMEM((w * d,), jnp.float32),
            pltpu.VMEM((h * d,), jnp.float32),
            pltpu.VMEM((32 * h * w,), jnp.float32),
            pltpu.VMEM_SHARED((2 * d * h * w,), jnp.float32),
            pltpu.SemaphoreType.DMA,
        ],
    )
    out = run(
        row_weight[:w].reshape(w * d),
        col_weight[:h].reshape(h * d),
    )
    return out.reshape(b, 2 * d, h, w)


# speedup vs baseline: 1.0090x; 1.0090x over previous
import jax
import jax.numpy as jnp
from jax import lax
from jax.experimental import pallas as pl
from jax.experimental.pallas import tpu as pltpu
from jax.experimental.pallas import tpu_sc as plsc


def _lane_take(v, idx):
    return v.at[idx].get(mode="promise_in_bounds")


def _transpose16(vs, iota):
    for s in (1, 2, 4, 8):
        perm = iota ^ s
        nv = []
        for i in range(16):
            pp = _lane_take(vs[i ^ s], perm)
            keep = (iota & s) == (i & s)
            nv.append(jnp.where(keep, vs[i], pp))
        vs = nv
    return vs


def _pos_embed_body(rw_hbm, cw_hbm, out_hbm, rw_v, cw_v, slab_v, pos_sh, sem):
    cid = lax.axis_index("c")
    sid = lax.axis_index("s")

    pltpu.sync_copy(rw_hbm, rw_v)
    pltpu.sync_copy(cw_hbm, cw_v)

    iota16 = lax.iota(jnp.int32, 16)


    @pl.when(sid < 8)
    def _build_x():
        c0 = sid * 32
        for g in range(2):
            cg = c0 + g * 16
            for xg in range(2):
                vs = [
                    rw_v[pl.ds((xg * 16 + x) * 256 + cg, 16)]
                    for x in range(16)
                ]
                t = _transpose16(vs, iota16)

                def row(y, _, t=t, g=g, xg=xg):
                    for j in range(16):
                        off = (g * 16 + j) * 1024 + y * 32 + xg * 16
                        slab_v[pl.ds(off, 16)] = t[j]
                    return 0

                lax.fori_loop(0, 32, row, 0)

    @pl.when(sid >= 8)
    def _build_y():
        c0 = sid * 32 - 256
        for g in range(2):
            cg = c0 + g * 16
            for yg in range(2):
                vs = [
                    cw_v[pl.ds((yg * 16 + y) * 256 + cg, 16)]
                    for y in range(16)
                ]
                t = _transpose16(vs, iota16)

                def row(y, _, t=t, g=g, yg=yg):
                    lane = jnp.full((16,), y, jnp.int32)
                    for j in range(16):
                        sp = _lane_take(t[j], lane)
                        off = (g * 16 + j) * 1024 + (yg * 16 + y) * 32
                        slab_v[pl.ds(off, 16)] = sp
                        slab_v[pl.ds(off + 16, 16)] = sp
                    return 0

                lax.fori_loop(0, 16, row, 0)

    pltpu.sync_copy(slab_v, pos_sh.at[pl.ds(sid * 32768, 32768)])
    plsc.subcore_barrier()

    half = (sid >> 3) * 262144
    batch = cid * 8 + (sid & 7)
    pltpu.async_copy(
        pos_sh.at[pl.ds(half, 262144)],
        out_hbm.at[batch, pl.ds(half, 262144)],
        sem,
    ).wait()


def kernel(uv_feat, row_weight, col_weight):
    b = uv_feat.shape[0]
    h, w = uv_feat.shape[-2], uv_feat.shape[-1]
    d = row_weight.shape[-1]
    assert (b, h, w, d) == (16, 32, 32, 256)

    mesh = plsc.VectorSubcoreMesh(core_axis_name="c", subcore_axis_name="s")
    run = pl.kernel(
        _pos_embed_body,
        mesh=mesh,
        out_type=jax.ShapeDtypeStruct((b, 2 * d * h * w), jnp.float32),
        scratch_types=[
            pltpu.VMEM((w * d,), jnp.float32),
            pltpu.VMEM((h * d,), jnp.float32),
            pltpu.VMEM((32 * h * w,), jnp.float32),
            pltpu.VMEM_SHARED((2 * d * h * w,), jnp.float32),
            pltpu.SemaphoreType.DMA,
        ],
    )
    out = run(
        row_weight[:w].reshape(w * d),
        col_weight[:h].reshape(h * d),
    )
    return out.reshape(b, 2 * d, h, w)


# speedup vs baseline: 2.2864x; 2.2659x over previous
import jax
import jax.numpy as jnp
from jax import lax
from jax.experimental import pallas as pl
from jax.experimental.pallas import tpu as pltpu
from jax.experimental.pallas import tpu_sc as plsc


def _lane_take(v, idx):
    return v.at[idx].get(mode="promise_in_bounds")


def _transpose16(vs, iota):
    for s in (1, 2, 4, 8):
        perm = iota ^ s
        nv = []
        for i in range(16):
            pp = _lane_take(vs[i ^ s], perm)
            keep = (iota & s) == (i & s)
            nv.append(jnp.where(keep, vs[i], pp))
        vs = nv
    return vs


def _sc_lookup_body(rw_hbm, cw_hbm, out_hbm, rw_v, cw_v, slab_v, sem):
    wid = lax.axis_index("s") * 2 + lax.axis_index("c")

    pltpu.sync_copy(rw_hbm, rw_v)
    pltpu.sync_copy(cw_hbm, cw_v)

    iota16 = lax.iota(jnp.int32, 16)

    def build(tab_v, c0):
        for g in range(2):
            vs = [
                tab_v[pl.ds((g * 16 + i) * 256 + c0, 16)] for i in range(16)
            ]
            t = _transpose16(vs, iota16)
            for j in range(16):
                slab_v[pl.ds(j * 32 + g * 16, 16)] = t[j]

    @pl.when(wid < 16)
    def _build_x():
        build(rw_v, wid * 16)

    @pl.when(wid >= 16)
    def _build_y():
        build(cw_v, wid * 16 - 256)

    pltpu.async_copy(slab_v, out_hbm.at[pl.ds(wid * 512, 512)], sem).wait()


def _tc_broadcast_body(ps_ref, o_ref):
    ps = ps_ref[...]
    xs = ps[:256]
    ys = ps[256:]
    x_tile = jnp.concatenate([xs] * 32, axis=1)
    y_tile = jnp.broadcast_to(
        ys[:, :, None], (256, 32, 32)).reshape(256, 1024)
    o_ref[0] = jnp.concatenate([x_tile, y_tile], axis=0)


def kernel(uv_feat, row_weight, col_weight):
    b = uv_feat.shape[0]
    h, w = uv_feat.shape[-2], uv_feat.shape[-1]
    d = row_weight.shape[-1]
    assert (b, h, w, d) == (16, 32, 32, 256)

    mesh = plsc.VectorSubcoreMesh(core_axis_name="c", subcore_axis_name="s")
    sc_lookup = pl.kernel(
        _sc_lookup_body,
        mesh=mesh,
        out_type=jax.ShapeDtypeStruct((2 * d * 32,), jnp.float32),
        scratch_types=[
            pltpu.VMEM((w * d,), jnp.float32),
            pltpu.VMEM((h * d,), jnp.float32),
            pltpu.VMEM((512,), jnp.float32),
            pltpu.SemaphoreType.DMA,
        ],
    )
    ps = sc_lookup(
        row_weight[:w].reshape(w * d),
        col_weight[:h].reshape(h * d),
    ).reshape(2 * d, 32)

    out = pl.pallas_call(
        _tc_broadcast_body,
        grid=(b,),
        in_specs=[pl.BlockSpec((2 * d, 32), lambda i: (0, 0))],
        out_specs=pl.BlockSpec((1, 2 * d, h * w), lambda i: (i, 0, 0)),
        out_shape=jax.ShapeDtypeStruct((b, 2 * d, h * w), jnp.float32),
    )(ps)
    return out.reshape(b, 2 * d, h, w)
